# Initial kernel scaffold; baseline (speedup 1.0000x reference)
import jax
import jax.numpy as jnp
from jax.experimental import pallas as pl


def kernel(x, edge_index, edge_weight, W1, b1, W2, b2):
    raise NotImplementedError("write your pallas kernel here")


# speedup vs baseline: 15.1333x; 15.1333x over previous
import functools

import jax
import jax.numpy as jnp
from jax import lax
from jax.experimental import pallas as pl
from jax.experimental.pallas import tpu as pltpu
from jax.experimental.pallas import tpu_sc as plsc

N = 10000
E = 320000
C = 128
NC = 2
NS = 16
L = 16
NW = NC * NS
EPW = E // NW
GB = 80
NG = EPW // GB
NPAD = 10240
RPT = NPAD // NS
BLK = 512
GRID = NPAD // BLK

_mesh = plsc.VectorSubcoreMesh(core_axis_name="c", subcore_axis_name="s")

_GDN = lax.GatherDimensionNumbers(
    offset_dims=(), collapsed_slice_dims=(0,), start_index_map=(0,))


def _bcast_lane(vec, j):
    idx = jnp.full((L, 1), j, jnp.int32)
    return lax.gather(vec, idx, _GDN, slice_sizes=(1,),
                      mode=lax.GatherScatterMode.PROMISE_IN_BOUNDS)


def _deg_body(dst_hbm, ew_hbm, out_hbm, dstv, ewv, zb, idxb, deg_sh):
    c = lax.axis_index("c")
    s = lax.axis_index("s")
    w = c * NS + s

    def zero(i, _):
        zb[pl.ds(i * L, L)] = jnp.zeros((L,), jnp.float32)
        return 0

    lax.fori_loop(0, RPT // L, zero, 0)
    pltpu.sync_copy(zb, deg_sh.at[pl.ds(s * RPT, RPT)])
    pltpu.sync_copy(dst_hbm.at[w], dstv)
    pltpu.sync_copy(ew_hbm.at[w], ewv)
    plsc.subcore_barrier()

    def body(g, _):
        idxb[...] = dstv[pl.ds(g * L, L)]
        pltpu.sync_copy(ewv.at[pl.ds(g * L, L)], deg_sh.at[idxb], add=True)
        return 0

    lax.fori_loop(0, EPW // L, body, 0)
    plsc.subcore_barrier()
    pltpu.sync_copy(deg_sh.at[pl.ds(s * RPT, RPT)],
                    out_hbm.at[c].at[pl.ds(s * RPT, RPT)])


_deg_call = pl.kernel(
    _deg_body,
    out_type=jax.ShapeDtypeStruct((NC, NPAD), jnp.float32),
    mesh=_mesh,
    scratch_types=[
        pltpu.VMEM((EPW,), jnp.int32),
        pltpu.VMEM((EPW,), jnp.float32),
        pltpu.VMEM((RPT,), jnp.float32),
        pltpu.VMEM((L,), jnp.int32),
        pltpu.VMEM_SHARED((NPAD,), jnp.float32),
    ],
)


def _agg_body(g_hbm, src_hbm, dst_hbm, ew_hbm, out_hbm,
              srcv, dstv, ewv, rows, srcidx, dstidx, acc):
    c = lax.axis_index("c")
    s = lax.axis_index("s")
    w = c * NS + s

    pltpu.sync_copy(src_hbm.at[w], srcv)
    pltpu.sync_copy(dst_hbm.at[w], dstv)
    pltpu.sync_copy(ew_hbm.at[w], ewv)

    for r in range(GB):
        for cc in range(C // L):
            rows[r, pl.ds(cc * L, L)] = jnp.zeros((L,), jnp.float32)

    def zloop(t, _):
        pltpu.sync_copy(rows, acc.at[pl.ds(s * RPT + t * GB, GB)])
        return 0

    lax.fori_loop(0, RPT // GB, zloop, 0)
    plsc.subcore_barrier()

    def body(g, _):
        base = g * GB
        for k in range(GB // L):
            srcidx[pl.ds(k * L, L)] = srcv[pl.ds(base + k * L, L)]
            dstidx[pl.ds(k * L, L)] = dstv[pl.ds(base + k * L, L)]
        pltpu.sync_copy(g_hbm.at[srcidx], rows)
        for k in range(GB // L):
            ewk = ewv[pl.ds(base + k * L, L)]
            for j in range(L):
                scale = _bcast_lane(ewk, j)
                r = k * L + j
                for cc in range(C // L):
                    rows[r, pl.ds(cc * L, L)] = rows[r, pl.ds(cc * L, L)] * scale
        pltpu.sync_copy(rows, acc.at[dstidx], add=True)
        return 0

    lax.fori_loop(0, NG, body, 0)
    plsc.subcore_barrier()
    pltpu.sync_copy(acc.at[pl.ds(s * RPT, RPT)],
                    out_hbm.at[c].at[pl.ds(s * RPT, RPT)])


_agg_call = pl.kernel(
    _agg_body,
    out_type=jax.ShapeDtypeStruct((NC, NPAD, C), jnp.float32),
    mesh=_mesh,
    scratch_types=[
        pltpu.VMEM((EPW,), jnp.int32),
        pltpu.VMEM((EPW,), jnp.int32),
        pltpu.VMEM((EPW,), jnp.float32),
        pltpu.VMEM((GB, C), jnp.float32),
        pltpu.VMEM((GB,), jnp.int32),
        pltpu.VMEM((GB,), jnp.int32),
        pltpu.VMEM_SHARED((NPAD, C), jnp.float32),
    ],
)


def _dis_from(degp_blk):
    deg = jnp.sum(degp_blk, axis=0) + 1.0
    return jnp.where(deg > 0, lax.rsqrt(deg), 0.0)


def _layer_in_body(x_ref, w_ref, degp_ref, h_ref, g_ref):
    h = jnp.dot(x_ref[...], w_ref[...], preferred_element_type=jnp.float32)
    dis = _dis_from(degp_ref[...])
    h_ref[...] = h
    g_ref[...] = h * dis[:, None]


_layer_in = pl.pallas_call(
    _layer_in_body,
    grid=(GRID,),
    in_specs=[
        pl.BlockSpec((BLK, C), lambda i: (i, 0)),
        pl.BlockSpec((C, C), lambda i: (0, 0)),
        pl.BlockSpec((NC, BLK), lambda i: (0, i)),
    ],
    out_specs=[pl.BlockSpec((BLK, C), lambda i: (i, 0))] * 2,
    out_shape=[jax.ShapeDtypeStruct((NPAD, C), jnp.float32)] * 2,
)


def _layer_mid_body(p_ref, h1_ref, degp_ref, b1_ref, w2_ref, h2_ref, g2_ref):
    agg = jnp.sum(p_ref[...], axis=0)
    dis = _dis_from(degp_ref[...])
    out1 = (agg * dis[:, None] + h1_ref[...] * (dis * dis)[:, None]
            + b1_ref[...][None, :])
    out1 = jnp.maximum(out1, 0.0)
    h2 = jnp.dot(out1, w2_ref[...], preferred_element_type=jnp.float32)
    h2_ref[...] = h2
    g2_ref[...] = h2 * dis[:, None]


_layer_mid = pl.pallas_call(
    _layer_mid_body,
    grid=(GRID,),
    in_specs=[
        pl.BlockSpec((NC, BLK, C), lambda i: (0, i, 0)),
        pl.BlockSpec((BLK, C), lambda i: (i, 0)),
        pl.BlockSpec((NC, BLK), lambda i: (0, i)),
        pl.BlockSpec((C,), lambda i: (0,)),
        pl.BlockSpec((C, C), lambda i: (0, 0)),
    ],
    out_specs=[pl.BlockSpec((BLK, C), lambda i: (i, 0))] * 2,
    out_shape=[jax.ShapeDtypeStruct((NPAD, C), jnp.float32)] * 2,
)


def _layer_out_body(p_ref, h2_ref, degp_ref, b2_ref, o_ref):
    agg = jnp.sum(p_ref[...], axis=0)
    dis = _dis_from(degp_ref[...])
    h = (agg * dis[:, None] + h2_ref[...] * (dis * dis)[:, None]
         + b2_ref[...][None, :])
    nrm = jnp.sqrt(jnp.sum(h * h, axis=1, keepdims=True))
    o_ref[...] = h / jnp.maximum(nrm, 1e-12)


_layer_out = pl.pallas_call(
    _layer_out_body,
    grid=(GRID,),
    in_specs=[
        pl.BlockSpec((NC, BLK, C), lambda i: (0, i, 0)),
        pl.BlockSpec((BLK, C), lambda i: (i, 0)),
        pl.BlockSpec((NC, BLK), lambda i: (0, i)),
        pl.BlockSpec((C,), lambda i: (0,)),
    ],
    out_specs=pl.BlockSpec((BLK, C), lambda i: (i, 0)),
    out_shape=jax.ShapeDtypeStruct((NPAD, C), jnp.float32),
)


def kernel(x, edge_index, edge_weight, W1, b1, W2, b2):
    ei = edge_index.astype(jnp.int32)
    src = ei[0].reshape(NW, EPW)
    dst = ei[1].reshape(NW, EPW)
    ew = edge_weight.astype(jnp.float32).reshape(NW, EPW)
    xp = jnp.pad(x, ((0, NPAD - N), (0, 0)))

    degp = _deg_call(dst, ew)
    h1, g1 = _layer_in(xp, W1, degp)
    p1 = _agg_call(g1, src, dst, ew)
    h2, g2 = _layer_mid(p1, h1, degp, b1, W2)
    p2 = _agg_call(g2, src, dst, ew)
    return _layer_out(p2, h2, degp, b2)[:N]


# speedup vs baseline: 28.7233x; 1.8980x over previous
import functools

import jax
import jax.numpy as jnp
from jax import lax
from jax.experimental import pallas as pl
from jax.experimental.pallas import tpu as pltpu
from jax.experimental.pallas import tpu_sc as plsc

N = 10000
E = 320000
C = 128
NC = 2
NS = 16
L = 16
NW = NC * NS
EPW = E // NW
GB = 80
NG = EPW // GB
NPAD = 10240
RPT = NPAD // NS
BLK = 512
GRID = NPAD // BLK

_mesh = plsc.VectorSubcoreMesh(core_axis_name="c", subcore_axis_name="s")

_GDN = lax.GatherDimensionNumbers(
    offset_dims=(), collapsed_slice_dims=(0,), start_index_map=(0,))


def _bcast_lane(vec, j):
    idx = jnp.full((L, 1), j, jnp.int32)
    return lax.gather(vec, idx, _GDN, slice_sizes=(1,),
                      mode=lax.GatherScatterMode.PROMISE_IN_BOUNDS)


def _deg_body(dst_hbm, ew_hbm, out_hbm, dstv, ewv, zb, idxb, deg_sh):
    c = lax.axis_index("c")
    s = lax.axis_index("s")
    w = c * NS + s

    def zero(i, _):
        zb[pl.ds(i * L, L)] = jnp.zeros((L,), jnp.float32)
        return 0

    lax.fori_loop(0, RPT // L, zero, 0)
    pltpu.sync_copy(zb, deg_sh.at[pl.ds(s * RPT, RPT)])
    pltpu.sync_copy(dst_hbm.at[w], dstv)
    pltpu.sync_copy(ew_hbm.at[w], ewv)
    plsc.subcore_barrier()

    def body(g, _):
        base = g * GB
        for k in range(GB // L):
            idxb[pl.ds(k * L, L)] = dstv[pl.ds(base + k * L, L)]
        pltpu.sync_copy(ewv.at[pl.ds(base, GB)], deg_sh.at[idxb], add=True)
        return 0

    lax.fori_loop(0, NG, body, 0)
    plsc.subcore_barrier()
    pltpu.sync_copy(deg_sh.at[pl.ds(s * RPT, RPT)],
                    out_hbm.at[c].at[pl.ds(s * RPT, RPT)])


_deg_call = pl.kernel(
    _deg_body,
    out_type=jax.ShapeDtypeStruct((NC, NPAD), jnp.float32),
    mesh=_mesh,
    scratch_types=[
        pltpu.VMEM((EPW,), jnp.int32),
        pltpu.VMEM((EPW,), jnp.float32),
        pltpu.VMEM((RPT,), jnp.float32),
        pltpu.VMEM((GB,), jnp.int32),
        pltpu.VMEM_SHARED((NPAD,), jnp.float32),
    ],
)


def _agg_body(g_hbm, src_hbm, dst_hbm, ew_hbm, out_hbm,
              ewv, r0, r1, r2, si0, si1, si2, di0, di1, di2, acc,
              gs0, gs1, gs2, ss0, ss1, ss2, is0, is1, is2, id0, id1, id2):
    c = lax.axis_index("c")
    s = lax.axis_index("s")
    w = c * NS + s
    rows = (r0, r1, r2)
    srcidx = (si0, si1, si2)
    dstidx = (di0, di1, di2)
    gsem = (gs0, gs1, gs2)
    ssem = (ss0, ss1, ss2)
    sisem = (is0, is1, is2)
    disem = (id0, id1, id2)

    pltpu.sync_copy(ew_hbm.at[w], ewv)

    def zrow(r, _):
        for cc in range(C // L):
            r0[r, pl.ds(cc * L, L)] = jnp.zeros((L,), jnp.float32)
        return 0

    lax.fori_loop(0, GB, zrow, 0)

    def zloop(t, _):
        pltpu.sync_copy(r0, acc.at[pl.ds(s * RPT + t * GB, GB)])
        return 0

    lax.fori_loop(0, RPT // GB, zloop, 0)

    def fire_src(g, b):
        pltpu.async_copy(src_hbm.at[w].at[g], srcidx[b], sisem[b])

    def wait_src(b):
        pltpu.make_async_copy(src_hbm.at[w].at[0], srcidx[b], sisem[b]).wait()

    def fire_dst(g, b):
        pltpu.async_copy(dst_hbm.at[w].at[g], dstidx[b], disem[b])

    def wait_dst(b):
        pltpu.make_async_copy(dst_hbm.at[w].at[0], dstidx[b], disem[b]).wait()

    def fire_gather(b):
        pltpu.async_copy(g_hbm.at[srcidx[b]], rows[b], gsem[b])

    def wait_gather(b):
        pltpu.make_async_copy(g_hbm.at[srcidx[b]], rows[b], gsem[b]).wait()

    def fire_scatter(b):
        pltpu.async_copy(rows[b], acc.at[dstidx[b]], ssem[b], add=True)

    def wait_scatter(b):
        pltpu.make_async_copy(rows[b], acc.at[dstidx[b]], ssem[b]).wait()

    def scale(g, b):
        rb = rows[b]

        def sk(k, _):
            ewk = ewv[pl.ds(g * GB + k * L, L)]
            for j in range(L):
                sc = _bcast_lane(ewk, j)
                r = k * L + j
                for cc in range(C // L):
                    rb[r, pl.ds(cc * L, L)] = rb[r, pl.ds(cc * L, L)] * sc
            return 0

        lax.fori_loop(0, GB // L, sk, 0)

    def substep(g, b, steady):
        b1 = (b + 1) % 3
        b2 = (b + 2) % 3
        wait_gather(b)
        scale(g, b)
        if steady:
            @pl.when(g >= 1)
            def _():
                wait_scatter(b2)

            @pl.when(g + 3 < NG)
            def _():
                fire_src(g + 3, b)
            fire_dst(g + 2, b2)
            wait_src(b2)
            fire_gather(b2)
        else:
            wait_scatter(b2)
        wait_dst(b)
        fire_scatter(b)

    fire_src(0, 0)
    fire_src(1, 1)
    fire_src(2, 2)
    fire_dst(0, 0)
    fire_dst(1, 1)
    wait_src(0)
    fire_gather(0)
    wait_src(1)
    fire_gather(1)
    plsc.subcore_barrier()

    def body(i, _):
        g = i * 3
        substep(g, 0, True)
        substep(g + 1, 1, True)
        substep(g + 2, 2, True)
        return 0

    lax.fori_loop(0, (NG - 2) // 3, body, 0)
    substep(NG - 2, 0, False)
    substep(NG - 1, 1, False)
    wait_scatter(1)
    plsc.subcore_barrier()
    pltpu.sync_copy(acc.at[pl.ds(s * RPT, RPT)],
                    out_hbm.at[c].at[pl.ds(s * RPT, RPT)])


_agg_call = pl.kernel(
    _agg_body,
    out_type=jax.ShapeDtypeStruct((NC, NPAD, C), jnp.float32),
    mesh=_mesh,
    scratch_types=(
        [pltpu.VMEM((EPW,), jnp.float32)]
        + [pltpu.VMEM((GB, C), jnp.float32)] * 3
        + [pltpu.VMEM((GB,), jnp.int32)] * 6
        + [pltpu.VMEM_SHARED((NPAD, C), jnp.float32)]
        + [pltpu.SemaphoreType.DMA] * 12
    ),
)


def _dis_from(degp_blk):
    deg = jnp.sum(degp_blk, axis=0) + 1.0
    return jnp.where(deg > 0, lax.rsqrt(deg), 0.0)


def _layer_in_body(x_ref, w_ref, degp_ref, h_ref, g_ref):
    h = jnp.dot(x_ref[...], w_ref[...], preferred_element_type=jnp.float32)
    dis = _dis_from(degp_ref[...])
    h_ref[...] = h
    g_ref[...] = h * dis[:, None]


_layer_in = pl.pallas_call(
    _layer_in_body,
    grid=(GRID,),
    in_specs=[
        pl.BlockSpec((BLK, C), lambda i: (i, 0)),
        pl.BlockSpec((C, C), lambda i: (0, 0)),
        pl.BlockSpec((NC, BLK), lambda i: (0, i)),
    ],
    out_specs=[pl.BlockSpec((BLK, C), lambda i: (i, 0))] * 2,
    out_shape=[jax.ShapeDtypeStruct((NPAD, C), jnp.float32)] * 2,
)


def _layer_mid_body(p_ref, h1_ref, degp_ref, b1_ref, w2_ref, h2_ref, g2_ref):
    agg = jnp.sum(p_ref[...], axis=0)
    dis = _dis_from(degp_ref[...])
    out1 = (agg * dis[:, None] + h1_ref[...] * (dis * dis)[:, None]
            + b1_ref[...][None, :])
    out1 = jnp.maximum(out1, 0.0)
    h2 = jnp.dot(out1, w2_ref[...], preferred_element_type=jnp.float32)
    h2_ref[...] = h2
    g2_ref[...] = h2 * dis[:, None]


_layer_mid = pl.pallas_call(
    _layer_mid_body,
    grid=(GRID,),
    in_specs=[
        pl.BlockSpec((NC, BLK, C), lambda i: (0, i, 0)),
        pl.BlockSpec((BLK, C), lambda i: (i, 0)),
        pl.BlockSpec((NC, BLK), lambda i: (0, i)),
        pl.BlockSpec((C,), lambda i: (0,)),
        pl.BlockSpec((C, C), lambda i: (0, 0)),
    ],
    out_specs=[pl.BlockSpec((BLK, C), lambda i: (i, 0))] * 2,
    out_shape=[jax.ShapeDtypeStruct((NPAD, C), jnp.float32)] * 2,
)


def _layer_out_body(p_ref, h2_ref, degp_ref, b2_ref, o_ref):
    agg = jnp.sum(p_ref[...], axis=0)
    dis = _dis_from(degp_ref[...])
    h = (agg * dis[:, None] + h2_ref[...] * (dis * dis)[:, None]
         + b2_ref[...][None, :])
    nrm = jnp.sqrt(jnp.sum(h * h, axis=1, keepdims=True))
    o_ref[...] = h / jnp.maximum(nrm, 1e-12)


_layer_out = pl.pallas_call(
    _layer_out_body,
    grid=(GRID,),
    in_specs=[
        pl.BlockSpec((NC, BLK, C), lambda i: (0, i, 0)),
        pl.BlockSpec((BLK, C), lambda i: (i, 0)),
        pl.BlockSpec((NC, BLK), lambda i: (0, i)),
        pl.BlockSpec((C,), lambda i: (0,)),
    ],
    out_specs=pl.BlockSpec((BLK, C), lambda i: (i, 0)),
    out_shape=jax.ShapeDtypeStruct((NPAD, C), jnp.float32),
)


def kernel(x, edge_index, edge_weight, W1, b1, W2, b2):
    ei = edge_index.astype(jnp.int32)
    src = ei[0].reshape(NW, NG, GB)
    dst = ei[1].reshape(NW, NG, GB)
    ew = edge_weight.astype(jnp.float32).reshape(NW, EPW)
    xp = jnp.pad(x, ((0, NPAD - N), (0, 0)))

    degp = _deg_call(dst.reshape(NW, EPW), ew)
    h1, g1 = _layer_in(xp, W1, degp)
    p1 = _agg_call(g1, src, dst, ew)
    h2, g2 = _layer_mid(p1, h1, degp, b1, W2)
    p2 = _agg_call(g2, src, dst, ew)
    return _layer_out(p2, h2, degp, b2)[:N]


# speedup vs baseline: 28.8250x; 1.0035x over previous
import functools

import jax
import jax.numpy as jnp
from jax import lax
from jax.experimental import pallas as pl
from jax.experimental.pallas import tpu as pltpu
from jax.experimental.pallas import tpu_sc as plsc

N = 10000
E = 320000
C = 128
NC = 2
NS = 16
L = 16
NW = NC * NS
EPW = E // NW
GB = 80
NG = EPW // GB
NPAD = 10240
RPT = NPAD // NS
BLK = 512
GRID = NPAD // BLK

_mesh = plsc.VectorSubcoreMesh(core_axis_name="c", subcore_axis_name="s")

_GDN = lax.GatherDimensionNumbers(
    offset_dims=(), collapsed_slice_dims=(0,), start_index_map=(0,))


def _bcast_lane(vec, j):
    idx = jnp.full((L, 1), j, jnp.int32)
    return lax.gather(vec, idx, _GDN, slice_sizes=(1,),
                      mode=lax.GatherScatterMode.PROMISE_IN_BOUNDS)


def _deg_body(dst_hbm, ew_hbm, out_hbm, dstv, ewv, zb, idxb, deg_sh):
    c = lax.axis_index("c")
    s = lax.axis_index("s")
    w = c * NS + s

    def zero(i, _):
        zb[pl.ds(i * L, L)] = jnp.zeros((L,), jnp.float32)
        return 0

    lax.fori_loop(0, RPT // L, zero, 0)
    pltpu.sync_copy(zb, deg_sh.at[pl.ds(s * RPT, RPT)])
    pltpu.sync_copy(dst_hbm.at[w], dstv)
    pltpu.sync_copy(ew_hbm.at[w], ewv)
    plsc.subcore_barrier()

    def body(g, _):
        base = g * GB
        for k in range(GB // L):
            idxb[pl.ds(k * L, L)] = dstv[pl.ds(base + k * L, L)]
        pltpu.sync_copy(ewv.at[pl.ds(base, GB)], deg_sh.at[idxb], add=True)
        return 0

    lax.fori_loop(0, NG, body, 0)
    plsc.subcore_barrier()
    pltpu.sync_copy(deg_sh.at[pl.ds(s * RPT, RPT)],
                    out_hbm.at[c].at[pl.ds(s * RPT, RPT)])


_deg_call = pl.kernel(
    _deg_body,
    out_type=jax.ShapeDtypeStruct((NC, NPAD), jnp.float32),
    mesh=_mesh,
    scratch_types=[
        pltpu.VMEM((EPW,), jnp.int32),
        pltpu.VMEM((EPW,), jnp.float32),
        pltpu.VMEM((RPT,), jnp.float32),
        pltpu.VMEM((GB,), jnp.int32),
        pltpu.VMEM_SHARED((NPAD,), jnp.float32),
    ],
)


def _agg_body(g_hbm, src_hbm, dst_hbm, ew_hbm, out_hbm,
              ewv, r0, r1, r2, si0, si1, si2, di0, di1, di2, acc,
              gs0, gs1, gs2, ss0, ss1, ss2, is0, is1, is2, id0, id1, id2):
    c = lax.axis_index("c")
    s = lax.axis_index("s")
    w = c * NS + s
    rows = (r0, r1, r2)
    srcidx = (si0, si1, si2)
    dstidx = (di0, di1, di2)
    gsem = (gs0, gs1, gs2)
    ssem = (ss0, ss1, ss2)
    sisem = (is0, is1, is2)
    disem = (id0, id1, id2)

    pltpu.sync_copy(ew_hbm.at[w], ewv)

    def zrow(r, _):
        for cc in range(C // L):
            r0[r, pl.ds(cc * L, L)] = jnp.zeros((L,), jnp.float32)
        return 0

    lax.fori_loop(0, GB, zrow, 0)

    def zloop(t, _):
        pltpu.sync_copy(r0, acc.at[pl.ds(s * RPT + t * GB, GB)])
        return 0

    lax.fori_loop(0, RPT // GB, zloop, 0)

    def fire_src(g, b):
        pltpu.async_copy(src_hbm.at[w].at[g], srcidx[b], sisem[b])

    def wait_src(b):
        pltpu.make_async_copy(src_hbm.at[w].at[0], srcidx[b], sisem[b]).wait()

    def fire_dst(g, b):
        pltpu.async_copy(dst_hbm.at[w].at[g], dstidx[b], disem[b])

    def wait_dst(b):
        pltpu.make_async_copy(dst_hbm.at[w].at[0], dstidx[b], disem[b]).wait()

    def fire_gather(b):
        pltpu.async_copy(g_hbm.at[srcidx[b]], rows[b], gsem[b])

    def wait_gather(b):
        pltpu.make_async_copy(g_hbm.at[srcidx[b]], rows[b], gsem[b]).wait()

    def fire_scatter(b):
        pltpu.async_copy(rows[b], acc.at[dstidx[b]], ssem[b], add=True)

    def wait_scatter(b):
        pltpu.make_async_copy(rows[b], acc.at[dstidx[b]], ssem[b]).wait()

    def scale(g, b):
        rb = rows[b]

        def sk(k, _):
            ewk = ewv[pl.ds(g * GB + k * L, L)]
            scs = [_bcast_lane(ewk, j) for j in range(L)]
            for j in range(L):
                r = k * L + j
                for cc in range(C // L):
                    rb[r, pl.ds(cc * L, L)] = rb[r, pl.ds(cc * L, L)] * scs[j]
            return 0

        lax.fori_loop(0, GB // L, sk, 0)

    def substep(g, b, steady):
        b1 = (b + 1) % 3
        b2 = (b + 2) % 3
        wait_gather(b)
        scale(g, b)
        if steady:
            @pl.when(g >= 1)
            def _():
                wait_scatter(b2)

            @pl.when(g + 3 < NG)
            def _():
                fire_src(g + 3, b)
            fire_dst(g + 2, b2)
            wait_src(b2)
            fire_gather(b2)
        else:
            wait_scatter(b2)
        wait_dst(b)
        fire_scatter(b)

    fire_src(0, 0)
    fire_src(1, 1)
    fire_src(2, 2)
    fire_dst(0, 0)
    fire_dst(1, 1)
    wait_src(0)
    fire_gather(0)
    wait_src(1)
    fire_gather(1)
    plsc.subcore_barrier()

    def body(i, _):
        g = i * 3
        substep(g, 0, True)
        substep(g + 1, 1, True)
        substep(g + 2, 2, True)
        return 0

    lax.fori_loop(0, (NG - 2) // 3, body, 0)
    substep(NG - 2, 0, False)
    substep(NG - 1, 1, False)
    wait_scatter(1)
    plsc.subcore_barrier()
    pltpu.sync_copy(acc.at[pl.ds(s * RPT, RPT)],
                    out_hbm.at[c].at[pl.ds(s * RPT, RPT)])


_agg_call = pl.kernel(
    _agg_body,
    out_type=jax.ShapeDtypeStruct((NC, NPAD, C), jnp.float32),
    mesh=_mesh,
    scratch_types=(
        [pltpu.VMEM((EPW,), jnp.float32)]
        + [pltpu.VMEM((GB, C), jnp.float32)] * 3
        + [pltpu.VMEM((GB,), jnp.int32)] * 6
        + [pltpu.VMEM_SHARED((NPAD, C), jnp.float32)]
        + [pltpu.SemaphoreType.DMA] * 12
    ),
)


def _dis_from(degp_blk):
    deg = jnp.sum(degp_blk, axis=0) + 1.0
    return jnp.where(deg > 0, lax.rsqrt(deg), 0.0)


def _layer_in_body(x_ref, w_ref, degp_ref, h_ref, g_ref):
    h = jnp.dot(x_ref[...], w_ref[...], preferred_element_type=jnp.float32)
    dis = _dis_from(degp_ref[...])
    h_ref[...] = h
    g_ref[...] = h * dis[:, None]


_layer_in = pl.pallas_call(
    _layer_in_body,
    grid=(GRID,),
    in_specs=[
        pl.BlockSpec((BLK, C), lambda i: (i, 0)),
        pl.BlockSpec((C, C), lambda i: (0, 0)),
        pl.BlockSpec((NC, BLK), lambda i: (0, i)),
    ],
    out_specs=[pl.BlockSpec((BLK, C), lambda i: (i, 0))] * 2,
    out_shape=[jax.ShapeDtypeStruct((NPAD, C), jnp.float32)] * 2,
)


def _layer_mid_body(p_ref, h1_ref, degp_ref, b1_ref, w2_ref, h2_ref, g2_ref):
    agg = jnp.sum(p_ref[...], axis=0)
    dis = _dis_from(degp_ref[...])
    out1 = (agg * dis[:, None] + h1_ref[...] * (dis * dis)[:, None]
            + b1_ref[...][None, :])
    out1 = jnp.maximum(out1, 0.0)
    h2 = jnp.dot(out1, w2_ref[...], preferred_element_type=jnp.float32)
    h2_ref[...] = h2
    g2_ref[...] = h2 * dis[:, None]


_layer_mid = pl.pallas_call(
    _layer_mid_body,
    grid=(GRID,),
    in_specs=[
        pl.BlockSpec((NC, BLK, C), lambda i: (0, i, 0)),
        pl.BlockSpec((BLK, C), lambda i: (i, 0)),
        pl.BlockSpec((NC, BLK), lambda i: (0, i)),
        pl.BlockSpec((C,), lambda i: (0,)),
        pl.BlockSpec((C, C), lambda i: (0, 0)),
    ],
    out_specs=[pl.BlockSpec((BLK, C), lambda i: (i, 0))] * 2,
    out_shape=[jax.ShapeDtypeStruct((NPAD, C), jnp.float32)] * 2,
)


def _layer_out_body(p_ref, h2_ref, degp_ref, b2_ref, o_ref):
    agg = jnp.sum(p_ref[...], axis=0)
    dis = _dis_from(degp_ref[...])
    h = (agg * dis[:, None] + h2_ref[...] * (dis * dis)[:, None]
         + b2_ref[...][None, :])
    nrm = jnp.sqrt(jnp.sum(h * h, axis=1, keepdims=True))
    o_ref[...] = h / jnp.maximum(nrm, 1e-12)


_layer_out = pl.pallas_call(
    _layer_out_body,
    grid=(GRID,),
    in_specs=[
        pl.BlockSpec((NC, BLK, C), lambda i: (0, i, 0)),
        pl.BlockSpec((BLK, C), lambda i: (i, 0)),
        pl.BlockSpec((NC, BLK), lambda i: (0, i)),
        pl.BlockSpec((C,), lambda i: (0,)),
    ],
    out_specs=pl.BlockSpec((BLK, C), lambda i: (i, 0)),
    out_shape=jax.ShapeDtypeStruct((N, C), jnp.float32),
)


def kernel(x, edge_index, edge_weight, W1, b1, W2, b2):
    ei = edge_index.astype(jnp.int32)
    src = ei[0].reshape(NW, NG, GB)
    dst = ei[1].reshape(NW, NG, GB)
    ew = edge_weight.astype(jnp.float32).reshape(NW, EPW)

    degp = _deg_call(dst.reshape(NW, EPW), ew)
    h1, g1 = _layer_in(x, W1, degp)
    p1 = _agg_call(g1, src, dst, ew)
    h2, g2 = _layer_mid(p1, h1, degp, b1, W2)
    p2 = _agg_call(g2, src, dst, ew)
    return _layer_out(p2, h2, degp, b2)
